# baseline (speedup 1.0000x reference)
import functools

import jax
import jax.numpy as jnp
from jax import lax
from jax.experimental import pallas as pl
from jax.experimental.pallas import tpu as pltpu
from jax.experimental.pallas import tpu_sc as plsc

N = 10000
E = 320000
D = 128

NC = 2
NS = 16
NW = NC * NS
B = 128
CHUNKS_PER_TILE = 80
E_PAD = NW * B * CHUNKS_PER_TILE
NPAD = 10112
ROWS_PER_SUB = NPAD // NS
G = 16
NBLK0 = 5
NBLK1 = 5


def _seg_sum_call(feat, src_p, dst_p, zacc, ones_blk, with_deg):
    mesh = plsc.VectorSubcoreMesh(core_axis_name="c", subcore_axis_name="s")
    outs = [jax.ShapeDtypeStruct((NC, NPAD, D), jnp.float32)]
    scratch = [
        pltpu.VMEM((2, G, 1, B), jnp.int32),
        pltpu.VMEM((2, G, 1, B), jnp.int32),
        pltpu.VMEM((2, B, D), jnp.float32),
        pltpu.VMEM_SHARED((NPAD, D), jnp.float32),
        [pltpu.SemaphoreType.DMA] * 2,
        [pltpu.SemaphoreType.DMA] * 2,
        pltpu.SemaphoreType.DMA,
    ]
    if with_deg:
        outs.append(jax.ShapeDtypeStruct((NC, NPAD, D), jnp.float32))

    def body(*refs):
        if with_deg:
            (feat_h, src_h, dst_h, zacc_h, ones_h,
             acc_out, deg_out,
             src_blk, dst_blk, rows_v, acc_sh, gsem, isem, ssem) = refs
        else:
            (feat_h, src_h, dst_h, zacc_h,
             acc_out,
             src_blk, dst_blk, rows_v, acc_sh, gsem, isem, ssem) = refs
        cid = lax.axis_index("c")
        sid = lax.axis_index("s")
        rz = ROWS_PER_SUB

        def zero_acc():
            pltpu.sync_copy(zacc_h.at[pl.ds(sid * rz, rz)],
                            acc_sh.at[pl.ds(sid * rz, rz)])

        def fire_idx_at(pbase, blk, slot):
            off = pbase + blk * G
            pltpu.async_copy(src_h.at[pl.ds(off, G)], src_blk.at[slot],
                             isem[slot])
            pltpu.async_copy(dst_h.at[pl.ds(off, G)], dst_blk.at[slot],
                             isem[slot])

        def wait_idx_at(pbase, blk, slot):
            off = pbase + blk * G
            pltpu.make_async_copy(src_h.at[pl.ds(off, G)], src_blk.at[slot],
                                  isem[slot]).wait()
            pltpu.make_async_copy(dst_h.at[pl.ds(off, G)], dst_blk.at[slot],
                                  isem[slot]).wait()

        def fire_gather(slot, b, j):
            pltpu.async_copy(feat_h.at[src_blk.at[slot, j, 0]],
                             rows_v.at[b], gsem[b])

        def wait_gather(slot, b, j):
            pltpu.make_async_copy(feat_h.at[src_blk.at[slot, j, 0]],
                                  rows_v.at[b], gsem[b]).wait()

        def sums_pass(pbase, nblk):
            fire_idx_p = lambda blk, slot: fire_idx_at(pbase, blk, slot)
            wait_idx_p = lambda blk, slot: wait_idx_at(pbase, blk, slot)
            fire_idx_p(0, 0)
            for blk in range(nblk):
                slot = blk & 1
                wait_idx_p(blk, slot)
                if blk + 1 < nblk:
                    fire_idx_p(blk + 1, slot ^ 1)
                fire_gather(slot, 0, 0)
                fire_gather(slot, 1, 1)

                @pl.loop(0, G // 2)
                def _(g):
                    for b in range(2):
                        j = g * 2 + b
                        wait_gather(slot, b, j)
                        pltpu.sync_copy(rows_v.at[b],
                                        acc_sh.at[dst_blk.at[slot, j, 0]],
                                        add=True)

                        @pl.when(j + 2 < G)
                        def _():
                            fire_gather(slot, b, j + 2)

        def deg_pass(pbase, nblk):
            fire_idx_p = lambda blk, slot: fire_idx_at(pbase, blk, slot)
            wait_idx_p = lambda blk, slot: wait_idx_at(pbase, blk, slot)
            fire_idx_p(0, 0)
            for blk in range(nblk):
                slot = blk & 1
                wait_idx_p(blk, slot)
                if blk + 1 < nblk:
                    fire_idx_p(blk + 1, slot ^ 1)

                @pl.loop(0, G)
                def _(j):
                    pltpu.async_copy(rows_v.at[0],
                                     acc_sh.at[dst_blk.at[slot, j, 0]],
                                     ssem, add=True)

                @pl.loop(0, G)
                def _(j):
                    pltpu.make_async_copy(rows_v.at[0],
                                          acc_sh.at[dst_blk.at[slot, j, 0]],
                                          ssem).wait()

        zero_acc()
        plsc.subcore_barrier()

        @pl.when(cid == 0)
        def _():
            sums_pass(sid * NBLK0 * G, NBLK0)

        @pl.when(cid == 1)
        def _():
            sums_pass(NS * NBLK0 * G + sid * NBLK1 * G, NBLK1)

        plsc.subcore_barrier()
        pltpu.sync_copy(acc_sh.at[pl.ds(sid * rz, rz)],
                        acc_out.at[cid, pl.ds(sid * rz, rz)])

        if with_deg:
            plsc.subcore_barrier()
            zero_acc()
            pltpu.sync_copy(ones_h, rows_v.at[0])
            plsc.subcore_barrier()

            @pl.when(cid == 0)
            def _():
                deg_pass(sid * NBLK0 * G, NBLK0)

            @pl.when(cid == 1)
            def _():
                deg_pass(NS * NBLK0 * G + sid * NBLK1 * G, NBLK1)

            plsc.subcore_barrier()
            pltpu.sync_copy(acc_sh.at[pl.ds(sid * rz, rz)],
                            deg_out.at[cid, pl.ds(sid * rz, rz)])

    k = pl.kernel(body, out_type=tuple(outs), mesh=mesh,
                  scratch_types=tuple(scratch))
    if with_deg:
        return k(feat, src_p, dst_p, zacc, ones_blk)
    return k(feat, src_p, dst_p, zacc)


_R = 400


def _tc_layer1(sums, deg, x, W_l, b_l, W_r):
    def body(s_ref, d_ref, x_ref, wl_ref, bl_ref, wr_ref, o_ref):
        s = s_ref[0] + s_ref[1]
        dg = d_ref[0, :, 0:1] + d_ref[1, :, 0:1]
        mean = s / jnp.maximum(dg, 1.0)
        acc = jnp.dot(mean, wl_ref[...], preferred_element_type=jnp.float32)
        acc = acc + jnp.dot(x_ref[...], wr_ref[...],
                            preferred_element_type=jnp.float32)
        o_ref[...] = jnp.maximum(acc + bl_ref[...], 0.0)

    return pl.pallas_call(
        body,
        grid=(N // _R,),
        in_specs=[
            pl.BlockSpec((NC, _R, D), lambda i: (0, i, 0)),
            pl.BlockSpec((NC, _R, D), lambda i: (0, i, 0)),
            pl.BlockSpec((_R, D), lambda i: (i, 0)),
            pl.BlockSpec((D, D), lambda i: (0, 0)),
            pl.BlockSpec((1, D), lambda i: (0, 0)),
            pl.BlockSpec((D, D), lambda i: (0, 0)),
        ],
        out_specs=pl.BlockSpec((_R, D), lambda i: (i, 0)),
        out_shape=jax.ShapeDtypeStruct((N, D), jnp.float32),
    )(sums, deg, x, W_l, b_l.reshape(1, D), W_r)


def _tc_layer2(sums, deg, h, W_l, b_l, W_r, W_lin, b_lin):
    def body(s_ref, d_ref, h_ref, wl_ref, bl_ref, wr_ref, wo_ref, bo_ref,
             o_ref):
        s = s_ref[0] + s_ref[1]
        dg = d_ref[0, :, 0:1] + d_ref[1, :, 0:1]
        mean = s / jnp.maximum(dg, 1.0)
        hv = h_ref[...]
        h2 = jnp.dot(mean, wl_ref[...], preferred_element_type=jnp.float32)
        h2 = h2 + jnp.dot(hv, wr_ref[...], preferred_element_type=jnp.float32)
        h3 = jnp.maximum(hv + h2 + bl_ref[...], 0.0)
        o_ref[...] = jnp.dot(h3, wo_ref[...],
                             preferred_element_type=jnp.float32) + bo_ref[...]

    return pl.pallas_call(
        body,
        grid=(N // _R,),
        in_specs=[
            pl.BlockSpec((NC, _R, D), lambda i: (0, i, 0)),
            pl.BlockSpec((NC, _R, D), lambda i: (0, i, 0)),
            pl.BlockSpec((_R, D), lambda i: (i, 0)),
            pl.BlockSpec((D, D), lambda i: (0, 0)),
            pl.BlockSpec((1, D), lambda i: (0, 0)),
            pl.BlockSpec((D, D), lambda i: (0, 0)),
            pl.BlockSpec((D, D), lambda i: (0, 0)),
            pl.BlockSpec((1, D), lambda i: (0, 0)),
        ],
        out_specs=pl.BlockSpec((_R, D), lambda i: (i, 0)),
        out_shape=jax.ShapeDtypeStruct((N, D), jnp.float32),
    )(sums, deg, h, W_l, b_l.reshape(1, D), W_r, W_lin, b_lin.reshape(1, D))


def kernel(x, edge_index, W_l1, b_l1, W_r1, W_l2, b_l2, W_r2, W_lin, b_lin):
    src = edge_index[0].astype(jnp.int32)
    dst = edge_index[1].astype(jnp.int32)
    npad = E_PAD - E
    src_p = jnp.concatenate([src, jnp.zeros((npad,), jnp.int32)])
    src_p = src_p.reshape(E_PAD // B, 1, B)
    dst_p = jnp.concatenate([dst, jnp.full((npad,), N, jnp.int32)])
    dst_p = dst_p.reshape(E_PAD // B, 1, B)
    zacc = jnp.zeros((NPAD, D), jnp.float32)
    ones_blk = jnp.ones((B, D), jnp.float32)

    sums1, deg = _seg_sum_call(x, src_p, dst_p, zacc, ones_blk, True)
    h = _tc_layer1(sums1, deg, x, W_l1, b_l1, W_r1)
    (sums2,) = _seg_sum_call(h, src_p, dst_p, zacc, None, False)
    out = _tc_layer2(sums2, deg, h, W_l2, b_l2, W_r2, W_lin, b_lin)
    return out


# speedup vs baseline: 1.3182x; 1.3182x over previous
import functools

import jax
import jax.numpy as jnp
from jax import lax
from jax.experimental import pallas as pl
from jax.experimental.pallas import tpu as pltpu
from jax.experimental.pallas import tpu_sc as plsc

N = 10000
E = 320000
D = 128

NC = 2
NS = 16
NW = NC * NS
B = 128
CHUNKS_PER_TILE = 80
E_PAD = NW * B * CHUNKS_PER_TILE
NPAD = 10112
ROWS_PER_SUB = NPAD // NS
G = 16
NBLK0 = 9
NBLK1 = 1


def _seg_sum_call(feat, src_p, dst_p, zacc, ones_blk, with_deg):
    mesh = plsc.VectorSubcoreMesh(core_axis_name="c", subcore_axis_name="s")
    outs = [jax.ShapeDtypeStruct((NC, NPAD, D), jnp.float32)]
    scratch = [
        pltpu.VMEM((2, G, 1, B), jnp.int32),
        pltpu.VMEM((2, G, 1, B), jnp.int32),
        pltpu.VMEM((2, B, D), jnp.float32),
        pltpu.VMEM_SHARED((NPAD, D), jnp.float32),
        [pltpu.SemaphoreType.DMA] * 2,
        [pltpu.SemaphoreType.DMA] * 2,
        pltpu.SemaphoreType.DMA,
    ]
    if with_deg:
        outs.append(jax.ShapeDtypeStruct((NC, NPAD, D), jnp.float32))

    def body(*refs):
        if with_deg:
            (feat_h, src_h, dst_h, zacc_h, ones_h,
             acc_out, deg_out,
             src_blk, dst_blk, rows_v, acc_sh, gsem, isem, ssem) = refs
        else:
            (feat_h, src_h, dst_h, zacc_h,
             acc_out,
             src_blk, dst_blk, rows_v, acc_sh, gsem, isem, ssem) = refs
        cid = lax.axis_index("c")
        sid = lax.axis_index("s")
        rz = ROWS_PER_SUB

        def zero_acc():
            pltpu.sync_copy(zacc_h.at[pl.ds(sid * rz, rz)],
                            acc_sh.at[pl.ds(sid * rz, rz)])

        def fire_idx_at(pbase, blk, slot):
            off = pbase + blk * G
            pltpu.async_copy(src_h.at[pl.ds(off, G)], src_blk.at[slot],
                             isem[slot])
            pltpu.async_copy(dst_h.at[pl.ds(off, G)], dst_blk.at[slot],
                             isem[slot])

        def wait_idx_at(pbase, blk, slot):
            off = pbase + blk * G
            pltpu.make_async_copy(src_h.at[pl.ds(off, G)], src_blk.at[slot],
                                  isem[slot]).wait()
            pltpu.make_async_copy(dst_h.at[pl.ds(off, G)], dst_blk.at[slot],
                                  isem[slot]).wait()

        def fire_gather(slot, b, j):
            pltpu.async_copy(feat_h.at[src_blk.at[slot, j, 0]],
                             rows_v.at[b], gsem[b])

        def wait_gather(slot, b, j):
            pltpu.make_async_copy(feat_h.at[src_blk.at[slot, j, 0]],
                                  rows_v.at[b], gsem[b]).wait()

        def sums_pass(pbase, nblk):
            fire_idx_p = lambda blk, slot: fire_idx_at(pbase, blk, slot)
            wait_idx_p = lambda blk, slot: wait_idx_at(pbase, blk, slot)
            fire_idx_p(0, 0)
            for blk in range(nblk):
                slot = blk & 1
                wait_idx_p(blk, slot)
                if blk + 1 < nblk:
                    fire_idx_p(blk + 1, slot ^ 1)
                fire_gather(slot, 0, 0)
                fire_gather(slot, 1, 1)

                @pl.loop(0, G // 2)
                def _(g):
                    for b in range(2):
                        j = g * 2 + b
                        wait_gather(slot, b, j)
                        pltpu.sync_copy(rows_v.at[b],
                                        acc_sh.at[dst_blk.at[slot, j, 0]],
                                        add=True)

                        @pl.when(j + 2 < G)
                        def _():
                            fire_gather(slot, b, j + 2)

        def deg_pass(pbase, nblk):
            fire_idx_p = lambda blk, slot: fire_idx_at(pbase, blk, slot)
            wait_idx_p = lambda blk, slot: wait_idx_at(pbase, blk, slot)
            fire_idx_p(0, 0)
            for blk in range(nblk):
                slot = blk & 1
                wait_idx_p(blk, slot)
                if blk + 1 < nblk:
                    fire_idx_p(blk + 1, slot ^ 1)

                @pl.loop(0, G)
                def _(j):
                    pltpu.async_copy(rows_v.at[0],
                                     acc_sh.at[dst_blk.at[slot, j, 0]],
                                     ssem, add=True)

                @pl.loop(0, G)
                def _(j):
                    pltpu.make_async_copy(rows_v.at[0],
                                          acc_sh.at[dst_blk.at[slot, j, 0]],
                                          ssem).wait()

        zero_acc()
        plsc.subcore_barrier()

        @pl.when(cid == 0)
        def _():
            sums_pass(sid * NBLK0 * G, NBLK0)

        @pl.when(cid == 1)
        def _():
            sums_pass(NS * NBLK0 * G + sid * NBLK1 * G, NBLK1)

        plsc.subcore_barrier()
        pltpu.sync_copy(acc_sh.at[pl.ds(sid * rz, rz)],
                        acc_out.at[cid, pl.ds(sid * rz, rz)])

        if with_deg:
            plsc.subcore_barrier()
            zero_acc()
            pltpu.sync_copy(ones_h, rows_v.at[0])
            plsc.subcore_barrier()

            @pl.when(cid == 0)
            def _():
                deg_pass(sid * NBLK0 * G, NBLK0)

            @pl.when(cid == 1)
            def _():
                deg_pass(NS * NBLK0 * G + sid * NBLK1 * G, NBLK1)

            plsc.subcore_barrier()
            pltpu.sync_copy(acc_sh.at[pl.ds(sid * rz, rz)],
                            deg_out.at[cid, pl.ds(sid * rz, rz)])

    k = pl.kernel(body, out_type=tuple(outs), mesh=mesh,
                  scratch_types=tuple(scratch))
    if with_deg:
        return k(feat, src_p, dst_p, zacc, ones_blk)
    return k(feat, src_p, dst_p, zacc)


_R = 400


def _tc_layer1(sums, deg, x, W_l, b_l, W_r):
    def body(s_ref, d_ref, x_ref, wl_ref, bl_ref, wr_ref, o_ref):
        s = s_ref[0] + s_ref[1]
        dg = d_ref[0, :, 0:1] + d_ref[1, :, 0:1]
        mean = s / jnp.maximum(dg, 1.0)
        acc = jnp.dot(mean, wl_ref[...], preferred_element_type=jnp.float32)
        acc = acc + jnp.dot(x_ref[...], wr_ref[...],
                            preferred_element_type=jnp.float32)
        o_ref[...] = jnp.maximum(acc + bl_ref[...], 0.0)

    return pl.pallas_call(
        body,
        grid=(N // _R,),
        in_specs=[
            pl.BlockSpec((NC, _R, D), lambda i: (0, i, 0)),
            pl.BlockSpec((NC, _R, D), lambda i: (0, i, 0)),
            pl.BlockSpec((_R, D), lambda i: (i, 0)),
            pl.BlockSpec((D, D), lambda i: (0, 0)),
            pl.BlockSpec((1, D), lambda i: (0, 0)),
            pl.BlockSpec((D, D), lambda i: (0, 0)),
        ],
        out_specs=pl.BlockSpec((_R, D), lambda i: (i, 0)),
        out_shape=jax.ShapeDtypeStruct((N, D), jnp.float32),
    )(sums, deg, x, W_l, b_l.reshape(1, D), W_r)


def _tc_layer2(sums, deg, h, W_l, b_l, W_r, W_lin, b_lin):
    def body(s_ref, d_ref, h_ref, wl_ref, bl_ref, wr_ref, wo_ref, bo_ref,
             o_ref):
        s = s_ref[0] + s_ref[1]
        dg = d_ref[0, :, 0:1] + d_ref[1, :, 0:1]
        mean = s / jnp.maximum(dg, 1.0)
        hv = h_ref[...]
        h2 = jnp.dot(mean, wl_ref[...], preferred_element_type=jnp.float32)
        h2 = h2 + jnp.dot(hv, wr_ref[...], preferred_element_type=jnp.float32)
        h3 = jnp.maximum(hv + h2 + bl_ref[...], 0.0)
        o_ref[...] = jnp.dot(h3, wo_ref[...],
                             preferred_element_type=jnp.float32) + bo_ref[...]

    return pl.pallas_call(
        body,
        grid=(N // _R,),
        in_specs=[
            pl.BlockSpec((NC, _R, D), lambda i: (0, i, 0)),
            pl.BlockSpec((NC, _R, D), lambda i: (0, i, 0)),
            pl.BlockSpec((_R, D), lambda i: (i, 0)),
            pl.BlockSpec((D, D), lambda i: (0, 0)),
            pl.BlockSpec((1, D), lambda i: (0, 0)),
            pl.BlockSpec((D, D), lambda i: (0, 0)),
            pl.BlockSpec((D, D), lambda i: (0, 0)),
            pl.BlockSpec((1, D), lambda i: (0, 0)),
        ],
        out_specs=pl.BlockSpec((_R, D), lambda i: (i, 0)),
        out_shape=jax.ShapeDtypeStruct((N, D), jnp.float32),
    )(sums, deg, h, W_l, b_l.reshape(1, D), W_r, W_lin, b_lin.reshape(1, D))


def kernel(x, edge_index, W_l1, b_l1, W_r1, W_l2, b_l2, W_r2, W_lin, b_lin):
    src = edge_index[0].astype(jnp.int32)
    dst = edge_index[1].astype(jnp.int32)
    npad = E_PAD - E
    src_p = jnp.concatenate([src, jnp.zeros((npad,), jnp.int32)])
    src_p = src_p.reshape(E_PAD // B, 1, B)
    dst_p = jnp.concatenate([dst, jnp.full((npad,), N, jnp.int32)])
    dst_p = dst_p.reshape(E_PAD // B, 1, B)
    zacc = jnp.zeros((NPAD, D), jnp.float32)
    ones_blk = jnp.ones((B, D), jnp.float32)

    sums1, deg = _seg_sum_call(x, src_p, dst_p, zacc, ones_blk, True)
    h = _tc_layer1(sums1, deg, x, W_l1, b_l1, W_r1)
    (sums2,) = _seg_sum_call(h, src_p, dst_p, zacc, None, False)
    out = _tc_layer2(sums2, deg, h, W_l2, b_l2, W_r2, W_lin, b_lin)
    return out
